# Initial kernel scaffold; baseline (speedup 1.0000x reference)
import jax
import jax.numpy as jnp
from jax.experimental import pallas as pl


def kernel(xyz, features, W1, g1, b1, m1, v1, W2, g2, b2, m2, v2, W3, bias3):
    raise NotImplementedError("write your pallas kernel here")


# speedup vs baseline: 5.8683x; 5.8683x over previous
import functools

import jax
import jax.numpy as jnp
from jax import lax
from jax.experimental import pallas as pl
from jax.experimental.pallas import tpu as pltpu
from jax.experimental.pallas import tpu_sc as plsc

_N = 16384
_P = 512
_S = 64
_RB = 128
_EPS = 1e-5
_B = 2
_CH = 19


def _score_topk_body(xyzT_ref, W1_ref, m1_ref, v1_ref, g1_ref, b1_ref,
                     W2_ref, m2_ref, v2_ref, g2_ref, b2_ref,
                     W3_ref, b3_ref, idx_ref):
    x = xyzT_ref[0]
    h = jnp.dot(W1_ref[...], x)
    h = (h - m1_ref[...]) / jnp.sqrt(v1_ref[...] + _EPS) * g1_ref[...] + b1_ref[...]
    h = jnp.maximum(h, 0.0)
    h = jnp.dot(W2_ref[...], h)
    h = (h - m2_ref[...]) / jnp.sqrt(v2_ref[...] + _EPS) * g2_ref[...] + b2_ref[...]
    h = jnp.maximum(h, 0.0)
    logits = jnp.dot(W3_ref[...], h) + b3_ref[...]
    p = jax.nn.sigmoid(logits)

    iota = lax.broadcasted_iota(jnp.int32, (_RB, _N), 1)
    idxs = []
    for _ in range(_S):
        m = jnp.max(p, axis=1, keepdims=True)
        cand = jnp.where(p == m, iota, _N)
        i_s = jnp.min(cand, axis=1)
        idxs.append(i_s)
        p = jnp.where(iota == i_s[:, None], -1.0, p)
    idx_ref[0] = jnp.stack(idxs, axis=1)


def _score_topk(xyzT, W1, g1, b1, m1, v1, W2, g2, b2, m2, v2, W3, bias3,
                interpret=False):
    col = lambda a: a.reshape(a.shape[0], 1)
    full = lambda a: pl.BlockSpec(a.shape, lambda b, j: (0,) * a.ndim)
    grid = (_B, _P // _RB)
    return pl.pallas_call(
        _score_topk_body,
        grid=grid,
        in_specs=[
            pl.BlockSpec((1, 3, _N), lambda b, j: (b, 0, 0)),
            full(W1), full(col(m1)), full(col(v1)), full(col(g1)), full(col(b1)),
            full(W2), full(col(m2)), full(col(v2)), full(col(g2)), full(col(b2)),
            pl.BlockSpec((_RB, 64), lambda b, j: (j, 0)),
            pl.BlockSpec((_RB, 1), lambda b, j: (j, 0)),
        ],
        out_specs=pl.BlockSpec((1, _RB, _S), lambda b, j: (b, j, 0)),
        out_shape=jax.ShapeDtypeStruct((_B, _P, _S), jnp.int32),
        interpret=interpret,
    )(xyzT, W1, col(m1), col(v1), col(g1), col(b1),
      W2, col(m2), col(v2), col(g2), col(b2), W3, col(bias3))


def _gather_sc(S, IDX):
    info = plsc.get_sparse_core_info()
    nc, ns = info.num_cores, info.num_subcores
    nw = nc * ns
    ntask = _B * _CH
    nrep = (ntask + nw - 1) // nw
    mesh = plsc.VectorSubcoreMesh(core_axis_name="c", subcore_axis_name="s")
    ps = _P * _S

    @functools.partial(
        pl.kernel, mesh=mesh,
        compiler_params=pltpu.CompilerParams(needs_layout_passes=False),
        out_type=jax.ShapeDtypeStruct((ntask, ps), jnp.float32),
        scratch_types=[
            pltpu.VMEM((_N,), jnp.float32),
            pltpu.VMEM((ps,), jnp.int32),
            pltpu.VMEM((ps,), jnp.float32),
        ],
    )
    def k(s_hbm, idx_hbm, out_hbm, tab_v, idx_v, out_v):
        wid = lax.axis_index("s") * nc + lax.axis_index("c")
        for rep in range(nrep):
            t = wid + rep * nw

            @pl.when(t < ntask)
            def _():
                pltpu.sync_copy(s_hbm.at[t], tab_v)
                pltpu.sync_copy(idx_hbm.at[t // _CH], idx_v)

                def body(i, carry):
                    ii = idx_v[pl.ds(i * 16, 16)]
                    out_v[pl.ds(i * 16, 16)] = plsc.load_gather(tab_v, [ii])
                    return carry

                lax.fori_loop(0, ps // 16, body, 0)
                pltpu.sync_copy(out_v, out_hbm.at[t])

    return k(S, IDX)


def kernel(xyz, features, W1, g1, b1, m1, v1, W2, g2, b2, m2, v2, W3, bias3):
    xyzT = jnp.transpose(xyz, (0, 2, 1))
    idx = _score_topk(xyzT, W1, g1, b1, m1, v1, W2, g2, b2, m2, v2, W3, bias3)
    S = jnp.concatenate([xyzT, features], axis=1).reshape(_B * _CH, _N)
    out = _gather_sc(S, idx.reshape(_B, _P * _S))
    return out.reshape(_B, _CH, _P, _S)


# speedup vs baseline: 6.5628x; 1.1183x over previous
import functools

import jax
import jax.numpy as jnp
from jax import lax
from jax.experimental import pallas as pl
from jax.experimental.pallas import tpu as pltpu
from jax.experimental.pallas import tpu_sc as plsc

_N = 16384
_P = 512
_S = 64
_RB = 128
_EPS = 1e-5
_B = 2
_CH = 19


def _score_topk_body(xyzT_ref, W1_ref, m1_ref, v1_ref, g1_ref, b1_ref,
                     W2_ref, m2_ref, v2_ref, g2_ref, b2_ref,
                     W3_ref, b3_ref, idx_ref):
    x = xyzT_ref[0]
    h = jnp.dot(W1_ref[...], x)
    h = (h - m1_ref[...]) / jnp.sqrt(v1_ref[...] + _EPS) * g1_ref[...] + b1_ref[...]
    h = jnp.maximum(h, 0.0)
    h = jnp.dot(W2_ref[...], h)
    h = (h - m2_ref[...]) / jnp.sqrt(v2_ref[...] + _EPS) * g2_ref[...] + b2_ref[...]
    h = jnp.maximum(h, 0.0)
    logits = jnp.dot(W3_ref[...], h) + b3_ref[...]
    p = jax.nn.sigmoid(logits)

    iota = lax.broadcasted_iota(jnp.int32, (_RB, _N), 1)
    q = _N // 4
    V = [p[:, k * q:(k + 1) * q] for k in range(4)]
    I = [iota[:, k * q:(k + 1) * q] for k in range(4)]

    def ce(a, b):
        keep = (V[a] > V[b]) | ((V[a] == V[b]) & (I[a] < I[b]))
        va = jnp.where(keep, V[a], V[b])
        ia = jnp.where(keep, I[a], I[b])
        vb = jnp.where(keep, V[b], V[a])
        ib = jnp.where(keep, I[b], I[a])
        V[a], I[a], V[b], I[b] = va, ia, vb, ib

    ce(0, 1); ce(2, 3); ce(0, 2); ce(1, 3); ce(1, 2)
    V1, V2, V3, V4 = V
    I1, I2, I3 = I[0], I[1], I[2]
    I4 = I[3]

    idxs = []
    for _ in range(_S):
        m = jnp.max(V1, axis=1, keepdims=True)
        cand = jnp.where(V1 == m, I1, _N)
        i_s = jnp.min(cand, axis=1)
        idxs.append(i_s)
        cond = I1 == i_s[:, None]
        V1 = jnp.where(cond, V2, V1)
        I1 = jnp.where(cond, I2, I1)
        V2 = jnp.where(cond, V3, V2)
        I2 = jnp.where(cond, I3, I2)
        V3 = jnp.where(cond, V4, V3)
        I3 = jnp.where(cond, I4, I3)
        V4 = jnp.where(cond, -1.0, V4)
    idx_ref[0] = jnp.stack(idxs, axis=1)


def _score_topk(xyzT, W1, g1, b1, m1, v1, W2, g2, b2, m2, v2, W3, bias3,
                interpret=False):
    col = lambda a: a.reshape(a.shape[0], 1)
    full = lambda a: pl.BlockSpec(a.shape, lambda b, j: (0,) * a.ndim)
    grid = (_B, _P // _RB)
    return pl.pallas_call(
        _score_topk_body,
        grid=grid,
        in_specs=[
            pl.BlockSpec((1, 3, _N), lambda b, j: (b, 0, 0)),
            full(W1), full(col(m1)), full(col(v1)), full(col(g1)), full(col(b1)),
            full(W2), full(col(m2)), full(col(v2)), full(col(g2)), full(col(b2)),
            pl.BlockSpec((_RB, 64), lambda b, j: (j, 0)),
            pl.BlockSpec((_RB, 1), lambda b, j: (j, 0)),
        ],
        out_specs=pl.BlockSpec((1, _RB, _S), lambda b, j: (b, j, 0)),
        out_shape=jax.ShapeDtypeStruct((_B, _P, _S), jnp.int32),
        interpret=interpret,
    )(xyzT, W1, col(m1), col(v1), col(g1), col(b1),
      W2, col(m2), col(v2), col(g2), col(b2), W3, col(bias3))


def _gather_sc(S, IDX):
    info = plsc.get_sparse_core_info()
    nc, ns = info.num_cores, info.num_subcores
    nw = nc * ns
    ntask = _B * _CH
    nrep = (ntask + nw - 1) // nw
    mesh = plsc.VectorSubcoreMesh(core_axis_name="c", subcore_axis_name="s")
    ps = _P * _S

    @functools.partial(
        pl.kernel, mesh=mesh,
        compiler_params=pltpu.CompilerParams(needs_layout_passes=False),
        out_type=jax.ShapeDtypeStruct((ntask, ps), jnp.float32),
        scratch_types=[
            pltpu.VMEM((_N,), jnp.float32),
            pltpu.VMEM((ps,), jnp.int32),
            pltpu.VMEM((ps,), jnp.float32),
        ],
    )
    def k(s_hbm, idx_hbm, out_hbm, tab_v, idx_v, out_v):
        wid = lax.axis_index("s") * nc + lax.axis_index("c")
        for rep in range(nrep):
            t = wid + rep * nw

            @pl.when(t < ntask)
            def _():
                pltpu.sync_copy(s_hbm.at[t], tab_v)
                pltpu.sync_copy(idx_hbm.at[t // _CH], idx_v)

                def body(i, carry):
                    ii = idx_v[pl.ds(i * 16, 16)]
                    out_v[pl.ds(i * 16, 16)] = plsc.load_gather(tab_v, [ii])
                    return carry

                lax.fori_loop(0, ps // 16, body, 0)
                pltpu.sync_copy(out_v, out_hbm.at[t])

    return k(S, IDX)


def kernel(xyz, features, W1, g1, b1, m1, v1, W2, g2, b2, m2, v2, W3, bias3):
    xyzT = jnp.transpose(xyz, (0, 2, 1))
    idx = _score_topk(xyzT, W1, g1, b1, m1, v1, W2, g2, b2, m2, v2, W3, bias3)
    S = jnp.concatenate([xyzT, features], axis=1).reshape(_B * _CH, _N)
    out = _gather_sc(S, idx.reshape(_B, _P * _S))
    return out.reshape(_B, _CH, _P, _S)
